# baseline (speedup 1.0000x reference)
import jax
import jax.numpy as jnp
from jax import lax
from jax.experimental import pallas as pl
from jax.experimental.pallas import tpu as pltpu
from jax.experimental.pallas import tpu_sc as plsc

D = 300
DP = 304
B = 4096
L = 200
NC = 2
NS = 16
NW = NC * NS
B_PER_W = B // NW
GRP = 4
OUT_W = D * L
DA = 160
DB = D - DA
N_DTA = DA // 16
N_DTB = (DP - DA) // 16

CHUNKS = (72, 64, 64)
CHUNK_OFF = (0, 72, 136)


def _body(x_hbm, wv_hbm, out_hbm, idx_v, ch0, ch1, ch2, half_a, half_b,
          gsem, wsem_a, wsem_b):
    wid = lax.axis_index("s") * NC + lax.axis_index("c")
    base = wid * B_PER_W
    iota = lax.iota(jnp.int32, 16)
    perms = [jnp.bitwise_and(iota + c, 15) for c in range(16)]
    bufs = (ch0, ch1, ch2)

    def stage_idx(b_first):
        pltpu.sync_copy(x_hbm.at[pl.ds((base + b_first) * L, GRP * L)], idx_v)

    def g_descs(b):
        s = (b % GRP) * L
        return [pltpu.make_async_copy(
            wv_hbm.at[idx_v.at[pl.ds(s + CHUNK_OFF[j], CHUNKS[j])]],
            bufs[j], gsem) for j in range(3)]

    def wa_desc(b):
        return pltpu.make_async_copy(
            half_a, out_hbm.at[pl.ds((base + b) * OUT_W, DA * L)], wsem_a)

    def wb_desc(b):
        return pltpu.make_async_copy(
            half_b.at[pl.ds(0, DB * L)],
            out_hbm.at[pl.ds((base + b) * OUT_W + DA * L, DB * L)], wsem_b)

    stage_idx(0)
    for g in g_descs(0):
        g.start()

    def transpose_half(dt0, n_dt, dest, d_base):
        def per_dt(t, c2):
            dt = dt0 + t
            dcol = dt * 16
            dloc = (dcol - d_base) * L
            for cid in range(3):
                buf = bufs[cid]
                ntiles = 5 if cid == 0 else 4
                maxl0 = CHUNKS[cid] - 16
                choff = CHUNK_OFF[cid]

                def per_tile(tt, c3):
                    loc_l0 = jnp.minimum(tt * 16, maxl0)
                    lrow = iota + loc_l0
                    sbase = dloc + choff + loc_l0 + iota
                    for c in range(16):
                        v = plsc.load_gather(
                            buf, [lrow, dcol + perms[c]])
                        plsc.store_scatter(
                            dest, [sbase + perms[c] * L], v)
                    return c3

                lax.fori_loop(0, ntiles, per_tile, 0)
            return c2
        lax.fori_loop(0, n_dt, per_dt, 0)

    def per_b(b, carry):
        for g in g_descs(b):
            g.wait()

        @pl.when(b > 0)
        def _():
            wa_desc(b - 1).wait()

        transpose_half(0, N_DTA, half_a, 0)
        wa_desc(b).start()

        @pl.when(b > 0)
        def _():
            wb_desc(b - 1).wait()

        transpose_half(N_DTA, N_DTB, half_b, DA)

        @pl.when(jnp.logical_and((b + 1) % GRP == 0, b + 1 < B_PER_W))
        def _():
            stage_idx(b + 1)

        @pl.when(b + 1 < B_PER_W)
        def _():
            for g in g_descs(b + 1):
                g.start()

        wb_desc(b).start()
        return carry

    lax.fori_loop(0, B_PER_W, per_b, 0)
    wa_desc(B_PER_W - 1).wait()
    wb_desc(B_PER_W - 1).wait()


_embed_transpose = pl.kernel(
    _body,
    out_type=jax.ShapeDtypeStruct((B * D * L,), jnp.float32),
    mesh=plsc.VectorSubcoreMesh(
        core_axis_name="c", subcore_axis_name="s",
        num_cores=NC, num_subcores=NS),
    compiler_params=pltpu.CompilerParams(
        use_tc_tiling_on_sc=False, needs_layout_passes=False,
        disable_bounds_checks=True),
    scratch_types=[
        pltpu.VMEM((GRP * L,), jnp.int32),
        pltpu.VMEM((CHUNKS[0], DP), jnp.float32),
        pltpu.VMEM((CHUNKS[1], DP), jnp.float32),
        pltpu.VMEM((CHUNKS[2], DP), jnp.float32),
        pltpu.VMEM((DA * L,), jnp.float32),
        pltpu.VMEM(((DP - DA) * L,), jnp.float32),
        pltpu.SemaphoreType.DMA,
        pltpu.SemaphoreType.DMA,
        pltpu.SemaphoreType.DMA,
    ],
)


def kernel(x, word_vectors):
    x32 = x.astype(jnp.int32).reshape(B * L)
    wvp = jnp.pad(word_vectors, ((0, 0), (0, DP - D)))
    return _embed_transpose(x32, wvp).reshape(B, D, L)
